# baseline (speedup 1.0000x reference)
import functools

import jax
import jax.numpy as jnp
from jax import lax
from jax.experimental import pallas as pl
from jax.experimental.pallas import tpu as pltpu
from jax.experimental.pallas import tpu_sc as plsc


B, N = 1024, 50
NC, NS, L = 2, 16, 16
NT_N = 7
NB = 4
BQ = B // NB
GROUPS = 8 * BQ // L

_W0, _B0, _AZ0, _TZ0, _PARAMS = 8, 88, 104, 864, 888

_mesh = plsc.VectorSubcoreMesh(core_axis_name="c", subcore_axis_name="s")


@functools.partial(
    pl.kernel,
    mesh=_mesh,
    out_type=jax.ShapeDtypeStruct((28, N, B), jnp.float32),
    compiler_params=pltpu.CompilerParams(needs_layout_passes=False),
    scratch_types=[
        pltpu.VMEM((7, 8, BQ), jnp.float32),
        pltpu.VMEM((8, BQ), jnp.float32),
        pltpu.VMEM((28, 8, BQ), jnp.float32),
        pltpu.VMEM((_PARAMS,), jnp.float32),
    ],
)
def _sc_kernel(info_hbm, mask_hbm, par_hbm, out_hbm, in_v, m_v, o_v, par_v):
    wid = lax.axis_index("s") * NC + lax.axis_index("c")

    @pl.when(wid < NT_N * NB)
    def _():
        tn = wid // NB
        n0 = tn * 8
        b0 = (wid % NB) * BQ
        pltpu.sync_copy(par_hbm, par_v)
        pltpu.sync_copy(info_hbm.at[:, pl.ds(n0, 8), pl.ds(b0, BQ)], in_v)
        pltpu.sync_copy(mask_hbm.at[pl.ds(n0, 8), pl.ds(b0, BQ)], m_v)

        def _splat(ref, idx):
            return plsc.load_gather(ref, [jnp.full((L,), idx, jnp.int32)])

        zero = jnp.zeros((L,), jnp.float32)
        JB = 4

        for jb in range(16 // JB):
            wv = [[_splat(par_v, _W0 + (jb * JB + jj) * 5 + k)
                   for k in range(5)] for jj in range(JB)]
            bv = [_splat(par_v, _B0 + jb * JB + jj) for jj in range(JB)]

            @plsc.parallel_loop(0, GROUPS, unroll=3)
            def _lin(g):
                r = g // (BQ // L)
                c = (g % (BQ // L)) * L
                m = m_v[r, pl.ds(c, L)]
                f = [in_v[k, r, pl.ds(c, L)] for k in range(5)]
                mf = jnp.where(m >= 0.5, m, zero)
                for jj in range(JB):
                    s = f[0] * wv[jj][0]
                    for k in range(1, 5):
                        s = s + f[k] * wv[jj][k]
                    y = s * m + bv[jj]
                    o_v[jb * JB + jj, r, pl.ds(c, L)] = (
                        jnp.maximum(y, 0.0) * mf)

        @plsc.parallel_loop(0, GROUPS, unroll=2)
        def _emb(g):
            r = g // (BQ // L)
            c = (g % (BQ // L)) * L
            m = m_v[r, pl.ds(c, L)]
            z = (in_v[5, r, pl.ds(c, L)] * m).astype(jnp.int32)
            t = (in_v[6, r, pl.ds(c, L)] * m).astype(jnp.int32)
            cond = (m >= 0.5) & (z >= 1) & (z <= 94)
            cf = jnp.where(cond, m, zero)
            zi = jnp.clip(z, 0, 94) * 8 + _AZ0
            for j in range(8):
                e = plsc.load_gather(par_v, [zi + j])
                o_v[16 + j, r, pl.ds(c, L)] = e * cf
            ti = jnp.clip(t, 0, 5) * 4 + _TZ0
            for j in range(4):
                e = plsc.load_gather(par_v, [ti + j])
                o_v[24 + j, r, pl.ds(c, L)] = e * cf

        pltpu.sync_copy(o_v, out_hbm.at[:, pl.ds(n0, 8), pl.ds(b0, BQ)])


@jax.jit
def kernel(elements_info, elements_mask, W, b, atom_embedding, type_embedding):
    eiT = jnp.transpose(elements_info, (2, 1, 0))
    mT = elements_mask.T
    par = jnp.concatenate([
        jnp.zeros((8,), jnp.float32), W.reshape(-1), b.reshape(-1),
        atom_embedding.reshape(-1), type_embedding.reshape(-1)])
    outT = _sc_kernel(eiT, mT, par)
    return jnp.transpose(outT, (2, 1, 0))


# speedup vs baseline: 1.0525x; 1.0525x over previous
import functools

import jax
import jax.numpy as jnp
from jax import lax
from jax.experimental import pallas as pl
from jax.experimental.pallas import tpu as pltpu
from jax.experimental.pallas import tpu_sc as plsc


B, N = 1024, 50
NC, NS, L = 2, 16, 16
NT_N = 7
NB = 4
BQ = B // NB
GROUPS = 8 * BQ // L

_W0, _B0, _AZ0, _TZ0, _PARAMS = 8, 88, 104, 864, 888

_mesh = plsc.VectorSubcoreMesh(core_axis_name="c", subcore_axis_name="s")


@functools.partial(
    pl.kernel,
    mesh=_mesh,
    out_type=jax.ShapeDtypeStruct((28, N, B), jnp.float32),
    compiler_params=pltpu.CompilerParams(needs_layout_passes=False),
    scratch_types=[
        pltpu.VMEM((7, 8, BQ), jnp.float32),
        pltpu.VMEM((8, BQ), jnp.float32),
        pltpu.VMEM((28, 8, BQ), jnp.float32),
        pltpu.VMEM((_PARAMS,), jnp.float32),
        pltpu.SemaphoreType.DMA,
    ],
)
def _sc_kernel(info_hbm, mask_hbm, par_hbm, out_hbm, in_v, m_v, o_v, par_v,
               sem):
    wid = lax.axis_index("s") * NC + lax.axis_index("c")

    @pl.when(wid < NT_N * NB)
    def _():
        tn = wid // NB
        n0 = tn * 8
        b0 = (wid % NB) * BQ
        pltpu.sync_copy(par_hbm, par_v)
        pltpu.sync_copy(info_hbm.at[:, pl.ds(n0, 8), pl.ds(b0, BQ)], in_v)
        pltpu.sync_copy(mask_hbm.at[pl.ds(n0, 8), pl.ds(b0, BQ)], m_v)

        def _splat(ref, idx):
            return plsc.load_gather(ref, [jnp.full((L,), idx, jnp.int32)])

        zero = jnp.zeros((L,), jnp.float32)
        copies = []
        JB = 4

        for jb in range(16 // JB):
            wv = [[_splat(par_v, _W0 + (jb * JB + jj) * 5 + k)
                   for k in range(5)] for jj in range(JB)]
            bv = [_splat(par_v, _B0 + jb * JB + jj) for jj in range(JB)]

            @plsc.parallel_loop(0, GROUPS, unroll=2)
            def _lin(g):
                r = g // (BQ // L)
                c = (g % (BQ // L)) * L
                m = m_v[r, pl.ds(c, L)]
                f = [in_v[k, r, pl.ds(c, L)] for k in range(5)]
                mf = jnp.where(m >= 0.5, m, zero)
                for jj in range(JB):
                    s = f[0] * wv[jj][0]
                    for k in range(1, 5):
                        s = s + f[k] * wv[jj][k]
                    y = s * m + bv[jj]
                    o_v[jb * JB + jj, r, pl.ds(c, L)] = (
                        jnp.maximum(y, 0.0) * mf)

            copies.append(pltpu.async_copy(
                o_v.at[pl.ds(jb * JB, JB)],
                out_hbm.at[pl.ds(jb * JB, JB), pl.ds(n0, 8), pl.ds(b0, BQ)],
                sem))

        @plsc.parallel_loop(0, GROUPS, unroll=2)
        def _emb(g):
            r = g // (BQ // L)
            c = (g % (BQ // L)) * L
            m = m_v[r, pl.ds(c, L)]
            z = (in_v[5, r, pl.ds(c, L)] * m).astype(jnp.int32)
            t = (in_v[6, r, pl.ds(c, L)] * m).astype(jnp.int32)
            cond = (m >= 0.5) & (z >= 1) & (z <= 94)
            cf = jnp.where(cond, m, zero)
            zi = jnp.clip(z, 0, 94) * 8 + _AZ0
            for j in range(8):
                e = plsc.load_gather(par_v, [zi + j])
                o_v[16 + j, r, pl.ds(c, L)] = e * cf
            ti = jnp.clip(t, 0, 5) * 4 + _TZ0
            for j in range(4):
                e = plsc.load_gather(par_v, [ti + j])
                o_v[24 + j, r, pl.ds(c, L)] = e * cf

        copies.append(pltpu.async_copy(
            o_v.at[pl.ds(16, 12)],
            out_hbm.at[pl.ds(16, 12), pl.ds(n0, 8), pl.ds(b0, BQ)], sem))
        for cp in copies:
            cp.wait()


@jax.jit
def kernel(elements_info, elements_mask, W, b, atom_embedding, type_embedding):
    eiT = jnp.transpose(elements_info, (2, 1, 0))
    mT = elements_mask.T
    par = jnp.concatenate([
        jnp.zeros((8,), jnp.float32), W.reshape(-1), b.reshape(-1),
        atom_embedding.reshape(-1), type_embedding.reshape(-1)])
    outT = _sc_kernel(eiT, mT, par)
    return jnp.transpose(outT, (2, 1, 0))
